# Initial kernel scaffold; baseline (speedup 1.0000x reference)
import jax
import jax.numpy as jnp
from jax.experimental import pallas as pl


def kernel(x):
    raise NotImplementedError("write your pallas kernel here")


# speedup vs baseline: 7.4021x; 7.4021x over previous
import functools

import jax
import jax.numpy as jnp
from jax import lax
from jax.experimental import pallas as pl

def _cumsum_lanes(a):
    rows, n = a.shape
    shift = 1
    while shift < n:
        z = jnp.zeros((rows, shift), a.dtype)
        a = a + jnp.concatenate([z, a[:, : n - shift]], axis=1)
        shift *= 2
    return a


def _median_body(x_ref, val_ref, idx_ref, *, kth):
    _INT_MIN = jnp.int32(-(2 ** 31))
    xb = x_ref[...]
    rows, n = xb.shape
    s = lax.bitcast_convert_type(xb, jnp.int32)
    key = jnp.where(s >= 0, s, jnp.bitwise_xor(jnp.bitwise_not(s), _INT_MIN))
    kth32 = jnp.int32(kth)

    def bit_step(i, p):
        bit = jnp.left_shift(jnp.int32(1), jnp.int32(31) - i)
        cand = jnp.bitwise_or(p, bit)
        trial = jnp.bitwise_xor(cand, _INT_MIN)
        c = jnp.sum((key < trial).astype(jnp.int32), axis=-1, keepdims=True)
        return jnp.where(c <= kth32, cand, p)

    p0 = jnp.zeros((rows, 1), jnp.int32)
    p = lax.fori_loop(0, 32, bit_step, p0)
    v = jnp.bitwise_xor(p, _INT_MIN)

    eq = (key == v)
    c_less = jnp.sum((key < v).astype(jnp.int32), axis=-1, keepdims=True)
    r = kth32 - c_less
    occ = _cumsum_lanes(eq.astype(jnp.int32))
    iota = lax.broadcasted_iota(jnp.int32, (rows, n), 1)
    hit = jnp.logical_and(eq, occ == r + 1)
    med_idx = jnp.sum(jnp.where(hit, iota, 0), axis=-1, keepdims=True)

    sv = jnp.where(v >= 0, v, jnp.bitwise_not(jnp.bitwise_xor(v, _INT_MIN)))
    val_ref[...] = lax.bitcast_convert_type(sv, jnp.float32)
    idx_ref[...] = med_idx


def _median_2d(x2, block_rows):
    m, n = x2.shape
    kth = (n - 1) // 2
    grid = (m // block_rows,)
    vals, idx = pl.pallas_call(
        functools.partial(_median_body, kth=kth),
        grid=grid,
        in_specs=[pl.BlockSpec((block_rows, n), lambda j: (j, 0))],
        out_specs=[
            pl.BlockSpec((block_rows, 1), lambda j: (j, 0)),
            pl.BlockSpec((block_rows, 1), lambda j: (j, 0)),
        ],
        out_shape=[
            jax.ShapeDtypeStruct((m, 1), jnp.float32),
            jax.ShapeDtypeStruct((m, 1), jnp.int32),
        ],
    )(x2)
    return vals[:, 0], idx[:, 0]


def kernel(x):
    b, s, n = x.shape
    m = b * s
    x2 = x.reshape(m, n)
    block_rows = 256 if m % 256 == 0 else m
    vals, idx = _median_2d(x2, block_rows)
    return vals.reshape(b, s), idx.reshape(b, s).astype(jnp.int64)


# speedup vs baseline: 9.8744x; 1.3340x over previous
import functools

import jax
import jax.numpy as jnp
from jax import lax
from jax.experimental import pallas as pl


def _cumsum_lanes(a):
    rows, n = a.shape
    shift = 1
    while shift < n:
        z = jnp.zeros((rows, shift), a.dtype)
        a = a + jnp.concatenate([z, a[:, : n - shift]], axis=1)
        shift *= 2
    return a


def _sum_lanes_i16(a):
    n = a.shape[1]
    while n > 128:
        n //= 2
        a = a[:, :n] + a[:, n:]
    return jnp.sum(a.astype(jnp.int32), axis=-1, keepdims=True)


def _greedy16(data, kth, limit):
    rows = data.shape[0]
    kth32 = jnp.int32(kth)

    def step(i, p):
        bit = jnp.left_shift(jnp.int32(1), jnp.int32(15) - i)
        cand = jnp.bitwise_or(p, bit)
        trial = (cand - 32768).astype(jnp.int16)
        c = _sum_lanes_i16((data < trial).astype(jnp.int16))
        return jnp.where(c + limit <= kth32, cand, p)

    p0 = jnp.zeros((rows, 1), jnp.int32)
    return lax.fori_loop(0, 16, step, p0)


def _median_body(x_ref, val_ref, idx_ref, *, kth):
    i32min = jnp.int32(-(2 ** 31))
    xb = x_ref[...]
    rows, n = xb.shape
    s = lax.bitcast_convert_type(xb, jnp.int32)
    key = jnp.where(s >= 0, s, jnp.bitwise_xor(jnp.bitwise_not(s), i32min))
    hi = jnp.right_shift(key, 16).astype(jnp.int16)
    lo = (jnp.bitwise_and(key, 0xFFFF) - 32768).astype(jnp.int16)

    hp = _greedy16(hi, kth, jnp.int32(0))
    hp16 = (hp - 32768).astype(jnp.int16)

    m = (hi == hp16)
    c_hi = _sum_lanes_i16((hi < hp16).astype(jnp.int16))
    lox = jnp.where(m, lo, jnp.int16(2 ** 15 - 1))
    lp = _greedy16(lox, kth, c_hi)
    lp16 = (lp - 32768).astype(jnp.int16)

    eq = jnp.logical_and(m, lo == lp16)
    lt = jnp.logical_or(hi < hp16, jnp.logical_and(m, lo < lp16))
    c_less = _sum_lanes_i16(lt.astype(jnp.int16))
    r16 = (jnp.int32(kth) + 1 - c_less).astype(jnp.int16)
    occ = _cumsum_lanes(eq.astype(jnp.int16))
    iota = lax.broadcasted_iota(jnp.int16, (rows, n), 1)
    hit = jnp.logical_and(eq, occ == r16)
    med_idx = _sum_lanes_i16(jnp.where(hit, iota, jnp.int16(0)))

    v = jnp.bitwise_or(jnp.left_shift(hp - 32768, 16), lp)
    sv = jnp.where(v >= 0, v, jnp.bitwise_not(jnp.bitwise_xor(v, i32min)))
    val_ref[...] = lax.bitcast_convert_type(sv, jnp.float32)
    idx_ref[...] = med_idx.astype(jnp.int32)


def _median_2d(x2, block_rows):
    m, n = x2.shape
    kth = (n - 1) // 2
    grid = (m // block_rows,)
    vals, idx = pl.pallas_call(
        functools.partial(_median_body, kth=kth),
        grid=grid,
        in_specs=[pl.BlockSpec((block_rows, n), lambda j: (j, 0))],
        out_specs=[
            pl.BlockSpec((block_rows, 1), lambda j: (j, 0)),
            pl.BlockSpec((block_rows, 1), lambda j: (j, 0)),
        ],
        out_shape=[
            jax.ShapeDtypeStruct((m, 1), jnp.float32),
            jax.ShapeDtypeStruct((m, 1), jnp.int32),
        ],
    )(x2)
    return vals[:, 0], idx[:, 0]


def kernel(x):
    b, s, n = x.shape
    m = b * s
    x2 = x.reshape(m, n)
    block_rows = 256 if m % 256 == 0 else m
    vals, idx = _median_2d(x2, block_rows)
    return vals.reshape(b, s), idx.reshape(b, s).astype(jnp.int64)
